# baseline (speedup 1.0000x reference)
import functools

import jax
import jax.numpy as jnp
from jax import lax
from jax.experimental import pallas as pl
from jax.experimental.pallas import tpu as pltpu
from jax.experimental.pallas import tpu_sc as plsc

N = 10000
E = 160000
D = 256
H = 4
HID = 128
SH = 2
NB = 10
BLK = N // NB
EPS = 1e-16


def _proj_body(x_ref, wq_ref, bq_ref, wk_ref, bk_ref, wv_ref, bv_ref,
               ws_ref, bs_ref, q_ref, k_ref, v_ref, s_ref):
    r = pl.program_id(0)
    x = x_ref[...]
    q_ref[...] = jnp.dot(x, wq_ref[0], preferred_element_type=jnp.float32) + bq_ref[pl.ds(r, 1), :]
    k_ref[...] = jnp.dot(x, wk_ref[0], preferred_element_type=jnp.float32) + bk_ref[pl.ds(r, 1), :]
    v_ref[...] = jnp.dot(x, wv_ref[0], preferred_element_type=jnp.float32) + bv_ref[pl.ds(r, 1), :]
    s_ref[...] = jnp.dot(x, ws_ref[0], preferred_element_type=jnp.float32) + bs_ref[pl.ds(r, 1), :]


def _projections(x, Wq, bq, Wk, bk, Wv, bv, Ws, bs):
    wide = pl.BlockSpec((1, D, H * D), lambda r, i: (r, 0, 0))
    bwide = pl.BlockSpec((2, H * D), lambda r, i: (0, 0))
    skinny = pl.BlockSpec((1, D, D), lambda r, i: (r, 0, 0))
    bskinny = pl.BlockSpec((2, D), lambda r, i: (0, 0))
    return pl.pallas_call(
        _proj_body,
        grid=(2, NB),
        in_specs=[
            pl.BlockSpec((BLK, D), lambda r, i: (i, 0)),
            wide, bwide, wide, bwide, wide, bwide, skinny, bskinny,
        ],
        out_specs=[
            pl.BlockSpec((BLK, H * D), lambda r, i: (r * NB + i, 0)),
            pl.BlockSpec((BLK, H * D), lambda r, i: (r * NB + i, 0)),
            pl.BlockSpec((BLK, H * D), lambda r, i: (r * NB + i, 0)),
            pl.BlockSpec((BLK, D), lambda r, i: (r * NB + i, 0)),
        ],
        out_shape=[
            jax.ShapeDtypeStruct((20480, H * D), jnp.float32),
            jax.ShapeDtypeStruct((2 * N, H * D), jnp.float32),
            jax.ShapeDtypeStruct((2 * N, H * D), jnp.float32),
            jax.ShapeDtypeStruct((2 * N, D), jnp.float32),
        ],
    )(x, Wq, bq, Wk, bk, Wv, bv, Ws, bs)


NTILE = 32
SCN = 20480
TR = SCN // NTILE
PN = 32
NPASS = TR // PN
CE = 1280
NCH = E // CE
GB = 8
SB = 512
MB = 1024
EE = E + SB
INVSQ = 0.0625


def _gather16(vec, idx16):
    dn = lax.GatherDimensionNumbers(offset_dims=(), collapsed_slice_dims=(0,),
                                    start_index_map=(0,))
    return lax.gather(vec, idx16[:, None], dn, slice_sizes=(1,),
                      mode=lax.GatherScatterMode.PROMISE_IN_BOUNDS)


def _bflysum(v, iota):
    for sh in (8, 4, 2, 1):
        v = v + _gather16(v, jnp.bitwise_xor(iota, sh))
    return v


def _edge_body(ed_hbm, q_hbm, k_hbm, v_hbm, num_hbm, den_hbm, med_hbm, mes_hbm,
               ebuf, stg_d, stg_s, md, ms, pend_d, pend_s,
               ka, kb, va, vb, qstage, accn, accd, sema, semb):
    wid = lax.axis_index("s") * 2 + lax.axis_index("c")
    base = wid * TR
    mybase = wid * EE
    iota = lax.iota(jnp.int32, 16)
    first4 = (iota < 4).astype(jnp.float32)
    oneh = [(iota == h).astype(jnp.float32) for h in range(H)]
    zeros16 = jnp.zeros((16,), jnp.float32)
    zeros16i = jnp.zeros((16,), jnp.int32)

    def p1_chunk(c, carry):
        sptr0, done0 = carry
        pltpu.sync_copy(ed_hbm.at[:, pl.ds(c * CE, CE)], ebuf)

        def p1_scan(g, sptr):
            off = g * 32
            vds = []
            for u in (0, 16):
                s16 = ebuf[0, pl.ds(off + u, 16)]
                d16 = ebuf[1, pl.ds(off + u, 16)]
                t16 = ebuf[2, pl.ds(off + u, 16)]
                vid_d = t16 * N + d16
                vid_s = t16 * N + s16
                m = jnp.logical_and(vid_d >= base, vid_d < base + TR)
                cnt = jnp.sum(m.astype(jnp.int32))
                vds.append((vid_d, vid_s, m, cnt))
            for vid_d, vid_s, m, cnt in vds:
                plsc.store_compressed(stg_d.at[pl.ds(sptr, 16)], vid_d, mask=m)
                plsc.store_compressed(stg_s.at[pl.ds(sptr, 16)], vid_s, mask=m)
                sptr = sptr + cnt
            return sptr

        sptr = lax.fori_loop(0, CE // 32, p1_scan, sptr0)
        blocks = lax.shift_right_logical(sptr, 9)

        def p1_flush(b, _):
            dst = pl.multiple_of(mybase + done0 + b * SB, SB)
            pltpu.sync_copy(stg_d.at[pl.ds(b * SB, SB)], med_hbm.at[pl.ds(dst, SB)])
            pltpu.sync_copy(stg_s.at[pl.ds(b * SB, SB)], mes_hbm.at[pl.ds(dst, SB)])
            return 0

        lax.fori_loop(0, blocks, p1_flush, 0)

        def p1_move(r, _):
            srcb = blocks * SB + r * 16
            stg_d[pl.ds(r * 16, 16)] = stg_d[pl.ds(srcb, 16)]
            stg_s[pl.ds(r * 16, 16)] = stg_s[pl.ds(srcb, 16)]
            return 0

        lax.fori_loop(0, SB // 16, p1_move, 0)
        return sptr - blocks * SB, done0 + blocks * SB

    sptr, done = lax.fori_loop(0, NCH, p1_chunk, (0, 0))
    fin = pl.multiple_of(mybase + done, SB)
    pltpu.sync_copy(stg_d.at[pl.ds(0, SB)], med_hbm.at[pl.ds(fin, SB)])
    pltpu.sync_copy(stg_s.at[pl.ds(0, SB)], mes_hbm.at[pl.ds(fin, SB)])
    cnt = done + sptr

    nch2 = lax.shift_right_logical(cnt + (MB - 1), 10)

    def start_kv(jb, kbuf, vbuf, sem):
        pltpu.async_copy(k_hbm.at[pend_s.at[pl.ds(jb, GB)]], kbuf, sem)
        pltpu.async_copy(v_hbm.at[pend_s.at[pl.ds(jb, GB)]], vbuf, sem)

    def wait_kv(jb, kbuf, vbuf, sem):
        pltpu.make_async_copy(k_hbm.at[pend_s.at[pl.ds(jb, GB)]], kbuf, sem).wait()
        pltpu.make_async_copy(v_hbm.at[pend_s.at[pl.ds(jb, GB)]], vbuf, sem).wait()

    def pass_body(p, _):
        lo = base + p * PN
        hi = lo + PN
        pltpu.sync_copy(q_hbm.at[pl.ds(lo, PN)], qstage)

        def zero_body(i, _):
            for t in range(64):
                accn[i, pl.ds(t * 16, 16)] = zeros16
            accd[i, pl.ds(0, 16)] = zeros16
            return 0

        lax.fori_loop(0, PN, zero_body, 0)

        def p2_chunk(c2, _):
            cb = c2 * MB
            src = pl.multiple_of(mybase + cb, MB)
            pltpu.sync_copy(med_hbm.at[pl.ds(src, MB)], md)
            pltpu.sync_copy(mes_hbm.at[pl.ds(src, MB)], ms)

            def p2_scan(g, ptr):
                off = g * 32
                parts = []
                for u in (0, 16):
                    d16 = md[pl.ds(off + u, 16)]
                    s16 = ms[pl.ds(off + u, 16)]
                    gv = (cb + off + u + iota) < cnt
                    m = jnp.logical_and(jnp.logical_and(d16 >= lo, d16 < hi), gv)
                    c16 = jnp.sum(m.astype(jnp.int32))
                    parts.append((d16, s16, m, c16))
                for d16, s16, m, c16 in parts:
                    plsc.store_compressed(pend_d.at[pl.ds(ptr, 16)], d16, mask=m)
                    plsc.store_compressed(pend_s.at[pl.ds(ptr, 16)], s16, mask=m)
                    ptr = ptr + c16
                return ptr

            ptr = lax.fori_loop(0, MB // 32, p2_scan, 0)
            pend_d[pl.ds(ptr, 16)] = zeros16i
            pend_s[pl.ds(ptr, 16)] = zeros16i
            nb = lax.shift_right_logical(ptr + (GB - 1), 3)

            def compute8(jb, kbuf, vbuf):
                ld16 = pend_d[pl.ds(jb, 16)] - lo

                def edge_body(e, _):
                    valid = (jb + e) < ptr
                    oe = (iota == e).astype(jnp.int32)
                    ld = jnp.sum(ld16 * oe)
                    ld = lax.select(valid, ld, 0)
                    av = zeros16
                    for h in range(H):
                        a16 = zeros16
                        for t in range(16):
                            cc = (h * 16 + t) * 16
                            a16 = a16 + qstage[ld, pl.ds(cc, 16)] * kbuf[e, pl.ds(cc, 16)]
                        av = av + oneh[h] * _bflysum(a16, iota)
                    ev = jnp.exp(av * INVSQ) * jnp.where(valid, 1.0, 0.0)
                    plsc.addupdate(accd.at[ld, pl.ds(0, 16)], ev * first4)
                    for h in range(H):
                        evh = _gather16(ev, jnp.full((16,), h, jnp.int32))
                        for t in range(16):
                            cc = (h * 16 + t) * 16
                            plsc.addupdate(accn.at[ld, pl.ds(cc, 16)],
                                           evh * vbuf[e, pl.ds(cc, 16)])
                    return 0

                lax.fori_loop(0, GB, edge_body, 0)

            @pl.when(nb > 0)
            def _():
                start_kv(0, ka, va, sema)

                def ring2(jj, _):
                    j0 = jj * 2
                    j1 = j0 + 1

                    @pl.when(j1 < nb)
                    def _():
                        start_kv(j1 * GB, kb, vb, semb)

                    wait_kv(j0 * GB, ka, va, sema)
                    compute8(j0 * GB, ka, va)

                    @pl.when(j1 < nb)
                    def _():
                        @pl.when(j1 + 1 < nb)
                        def _():
                            start_kv((j1 + 1) * GB, ka, va, sema)

                        wait_kv(j1 * GB, kb, vb, semb)
                        compute8(j1 * GB, kb, vb)

                    return 0

                lax.fori_loop(0, lax.shift_right_logical(nb + 1, 1), ring2, 0)

            return 0

        lax.fori_loop(0, nch2, p2_chunk, 0)
        pltpu.sync_copy(accn, num_hbm.at[pl.ds(lo, PN)])
        pltpu.sync_copy(accd, den_hbm.at[pl.ds(lo, PN)])
        return 0

    lax.fori_loop(0, NPASS, pass_body, 0)


def _edge_phase(q2, k2, v2, edge_index, edge_type):
    ed = jnp.concatenate([edge_index, edge_type.reshape(1, E)], axis=0)
    mesh = plsc.VectorSubcoreMesh(core_axis_name="c", subcore_axis_name="s")
    num, den, _, _ = pl.kernel(
        _edge_body,
        mesh=mesh,
        compiler_params=pltpu.CompilerParams(needs_layout_passes=False),
        out_type=[
            jax.ShapeDtypeStruct((SCN, H * D), jnp.float32),
            jax.ShapeDtypeStruct((SCN, 16), jnp.float32),
            jax.ShapeDtypeStruct((NTILE * EE,), jnp.int32),
            jax.ShapeDtypeStruct((NTILE * EE,), jnp.int32),
        ],
        scratch_types=[
            pltpu.VMEM((3, CE), jnp.int32),
            pltpu.VMEM((CE + 2 * SB + 16,), jnp.int32),
            pltpu.VMEM((CE + 2 * SB + 16,), jnp.int32),
            pltpu.VMEM((MB,), jnp.int32),
            pltpu.VMEM((MB,), jnp.int32),
            pltpu.VMEM((MB + 16,), jnp.int32),
            pltpu.VMEM((MB + 16,), jnp.int32),
            pltpu.VMEM((GB, H * D), jnp.float32),
            pltpu.VMEM((GB, H * D), jnp.float32),
            pltpu.VMEM((GB, H * D), jnp.float32),
            pltpu.VMEM((GB, H * D), jnp.float32),
            pltpu.VMEM((PN, H * D), jnp.float32),
            pltpu.VMEM((PN, H * D), jnp.float32),
            pltpu.VMEM((PN, 16), jnp.float32),
            pltpu.SemaphoreType.DMA,
            pltpu.SemaphoreType.DMA,
        ],
    )(ed, q2, k2, v2)
    return num, den


def _combine_body(num0_ref, den0_ref, s0_ref, num1_ref, den1_ref, s1_ref,
                  x_ref, wgu_ref, wgx_ref, bg_ref, a1_ref, b1_ref, a2_ref,
                  z0_ref, z1_ref, w_ref):
    i = pl.program_id(0)
    x = x_ref[...]
    xg = jnp.dot(x, wgx_ref[...], preferred_element_type=jnp.float32)

    @pl.when(i == 0)
    def _():
        w_ref[...] = jnp.zeros_like(w_ref)

    wacc = jnp.zeros((SH, 2), jnp.float32)
    for r, (num_ref, den_ref, s_ref, z_ref) in enumerate(
            ((num0_ref, den0_ref, s0_ref, z0_ref),
             (num1_ref, den1_ref, s1_ref, z1_ref))):
        num = num_ref[...]
        den = den_ref[...]
        u = jnp.zeros((BLK, D), jnp.float32)
        for h in range(H):
            u = u + num[:, h * D:(h + 1) * D] / (den[:, h:h + 1] + EPS)
        u = u * (1.0 / H) + s_ref[...]
        g = jax.nn.sigmoid(jnp.dot(u, wgu_ref[...], preferred_element_type=jnp.float32)
                           + xg + bg_ref[...])
        z = jnp.tanh(u) * g + x * (1.0 - g)
        z_ref[...] = z
        for sh in range(SH):
            t = jnp.tanh(jnp.dot(z, a1_ref[sh], preferred_element_type=jnp.float32)
                         + b1_ref[sh:sh + 1, :])
            t = jnp.dot(t, a2_ref[sh], preferred_element_type=jnp.float32)
            wacc = wacc + jnp.sum(t) * (jnp.arange(SH)[:, None] == sh) * \
                (jnp.arange(2)[None, :] == r)
    w_ref[...] += wacc


def _combine(num, den, skip, x, Wg, bg, A1, b1, A2):
    Wgu = Wg[:D]
    Wgx = Wg[D:]
    bg2 = bg.reshape(1, D)
    full = lambda *s: pl.BlockSpec(s, lambda i: (0,) * len(s))
    half = lambda r: pl.BlockSpec((BLK, H * D), lambda i, _r=r: (_r * NB + i, 0))
    halfd = lambda r: pl.BlockSpec((BLK, 16), lambda i, _r=r: (_r * NB + i, 0))
    halfs = lambda r: pl.BlockSpec((BLK, D), lambda i, _r=r: (_r * NB + i, 0))
    return pl.pallas_call(
        _combine_body,
        grid=(NB,),
        in_specs=[
            half(0), halfd(0), halfs(0), half(1), halfd(1), halfs(1),
            pl.BlockSpec((BLK, D), lambda i: (i, 0)),
            full(D, D), full(D, D), full(1, D),
            full(SH, D, HID), full(SH, HID), full(SH, HID, 1),
        ],
        out_specs=[
            pl.BlockSpec((BLK, D), lambda i: (i, 0)),
            pl.BlockSpec((BLK, D), lambda i: (i, 0)),
            pl.BlockSpec((SH, 2), lambda i: (0, 0)),
        ],
        out_shape=[
            jax.ShapeDtypeStruct((N, D), jnp.float32),
            jax.ShapeDtypeStruct((N, D), jnp.float32),
            jax.ShapeDtypeStruct((SH, 2), jnp.float32),
        ],
    )(num, den, skip, num, den, skip, x, Wgu, Wgx, bg2, A1, b1, A2)


def _final_body(z0_ref, z1_ref, w_ref, o_ref):
    w = w_ref[...] * (1.0 / N)
    ew = jnp.exp(w - jnp.max(w, axis=1, keepdims=True))
    beta = ew / jnp.sum(ew, axis=1, keepdims=True)
    c0 = jnp.sum(beta[:, 0]) * (1.0 / SH)
    c1 = jnp.sum(beta[:, 1]) * (1.0 / SH)
    o_ref[...] = c0 * z0_ref[...] + c1 * z1_ref[...]


def _final(z0, z1, w):
    return pl.pallas_call(
        _final_body,
        grid=(NB,),
        in_specs=[
            pl.BlockSpec((BLK, D), lambda i: (i, 0)),
            pl.BlockSpec((BLK, D), lambda i: (i, 0)),
            pl.BlockSpec((SH, 2), lambda i: (0, 0)),
        ],
        out_specs=pl.BlockSpec((BLK, D), lambda i: (i, 0)),
        out_shape=jax.ShapeDtypeStruct((N, D), jnp.float32),
    )(z0, z1, w)


def kernel(features, edge_index, edge_type, Wq, bq, Wk, bk, Wv, bv, Ws, bs,
           Wg, bg, A1, b1, A2):
    q2, k2, v2, skip = _projections(features, Wq, bq, Wk, bk, Wv, bv, Ws, bs)
    num, den = _edge_phase(q2, k2, v2, edge_index, edge_type)
    z0, z1, w = _combine(num, den, skip, features, Wg, bg, A1, b1, A2)
    return _final(z0, z1, w)


# speedup vs baseline: 1.0047x; 1.0047x over previous
import jax
import jax.numpy as jnp
from jax import lax
from jax.experimental import pallas as pl
from jax.experimental.pallas import tpu as pltpu
from jax.experimental.pallas import tpu_sc as plsc

N = 10000
E = 160000
D = 256
H = 4
HID = 128
SH = 2
NB = 10
BLK = N // NB
EPS = 1e-16


def _proj_body(x_ref, wq_ref, bq_ref, wk_ref, bk_ref, wv_ref, bv_ref,
               ws_ref, bs_ref, q_ref, k_ref, v_ref, s_ref):
    r = pl.program_id(0)
    x = x_ref[...]
    q_ref[...] = jnp.dot(x, wq_ref[0], preferred_element_type=jnp.float32) + bq_ref[pl.ds(r, 1), :]
    k_ref[...] = jnp.dot(x, wk_ref[0], preferred_element_type=jnp.float32) + bk_ref[pl.ds(r, 1), :]
    v_ref[...] = jnp.dot(x, wv_ref[0], preferred_element_type=jnp.float32) + bv_ref[pl.ds(r, 1), :]
    s_ref[...] = jnp.dot(x, ws_ref[0], preferred_element_type=jnp.float32) + bs_ref[pl.ds(r, 1), :]


def _projections(x, Wq, bq, Wk, bk, Wv, bv, Ws, bs):
    wide = pl.BlockSpec((1, D, H * D), lambda r, i: (r, 0, 0))
    bwide = pl.BlockSpec((2, H * D), lambda r, i: (0, 0))
    skinny = pl.BlockSpec((1, D, D), lambda r, i: (r, 0, 0))
    bskinny = pl.BlockSpec((2, D), lambda r, i: (0, 0))
    return pl.pallas_call(
        _proj_body,
        grid=(2, NB),
        in_specs=[
            pl.BlockSpec((BLK, D), lambda r, i: (i, 0)),
            wide, bwide, wide, bwide, wide, bwide, skinny, bskinny,
        ],
        out_specs=[
            pl.BlockSpec((BLK, H * D), lambda r, i: (r * NB + i, 0)),
            pl.BlockSpec((BLK, H * D), lambda r, i: (r * NB + i, 0)),
            pl.BlockSpec((BLK, H * D), lambda r, i: (r * NB + i, 0)),
            pl.BlockSpec((BLK, D), lambda r, i: (r * NB + i, 0)),
        ],
        out_shape=[
            jax.ShapeDtypeStruct((20480, H * D), jnp.float32),
            jax.ShapeDtypeStruct((2 * N, H * D), jnp.float32),
            jax.ShapeDtypeStruct((2 * N, H * D), jnp.float32),
            jax.ShapeDtypeStruct((2 * N, D), jnp.float32),
        ],
    )(x, Wq, bq, Wk, bk, Wv, bv, Ws, bs)


NTILE = 32
SCN = 20480
TR = SCN // NTILE
PN = 32
NPASS = TR // PN
CE = 1280
NCH = E // CE
GB = 8
SB = 512
MB = 1024
EE = E + SB
INVSQ = 0.0625


def _gather16(vec, idx16):
    dn = lax.GatherDimensionNumbers(offset_dims=(), collapsed_slice_dims=(0,),
                                    start_index_map=(0,))
    return lax.gather(vec, idx16[:, None], dn, slice_sizes=(1,),
                      mode=lax.GatherScatterMode.PROMISE_IN_BOUNDS)


def _bflysum(v, iota):
    for sh in (8, 4, 2, 1):
        v = v + _gather16(v, jnp.bitwise_xor(iota, sh))
    return v


def _edge_body(ed_hbm, q_hbm, k_hbm, v_hbm, num_hbm, den_hbm, med_hbm, mes_hbm,
               ebuf, stg_d, stg_s, md, ms, pend_d, pend_s,
               ka, kb, va, vb, qstage, accn, accd, sema, semb):
    wid = lax.axis_index("s") * 2 + lax.axis_index("c")
    base = wid * TR
    mybase = wid * EE
    iota = lax.iota(jnp.int32, 16)
    first4 = (iota < 4).astype(jnp.float32)
    oneh = [(iota == h).astype(jnp.float32) for h in range(H)]
    zeros16 = jnp.zeros((16,), jnp.float32)
    zeros16i = jnp.zeros((16,), jnp.int32)

    def p1_chunk(c, carry):
        sptr0, done0 = carry
        pltpu.sync_copy(ed_hbm.at[:, pl.ds(c * CE, CE)], ebuf)

        def p1_scan(g, sptr):
            off = g * 16
            s16 = ebuf[0, pl.ds(off, 16)]
            d16 = ebuf[1, pl.ds(off, 16)]
            t16 = ebuf[2, pl.ds(off, 16)]
            vid_d = t16 * N + d16
            vid_s = t16 * N + s16
            m = jnp.logical_and(vid_d >= base, vid_d < base + TR)
            cnt = jnp.sum(m.astype(jnp.int32))
            plsc.store_compressed(stg_d.at[pl.ds(sptr, 16)], vid_d, mask=m)
            plsc.store_compressed(stg_s.at[pl.ds(sptr, 16)], vid_s, mask=m)
            return sptr + cnt

        sptr = lax.fori_loop(0, CE // 16, p1_scan, sptr0)
        blocks = lax.shift_right_logical(sptr, 9)

        def p1_flush(b, _):
            dst = pl.multiple_of(mybase + done0 + b * SB, SB)
            pltpu.sync_copy(stg_d.at[pl.ds(b * SB, SB)], med_hbm.at[pl.ds(dst, SB)])
            pltpu.sync_copy(stg_s.at[pl.ds(b * SB, SB)], mes_hbm.at[pl.ds(dst, SB)])
            return 0

        lax.fori_loop(0, blocks, p1_flush, 0)

        def p1_move(r, _):
            srcb = blocks * SB + r * 16
            stg_d[pl.ds(r * 16, 16)] = stg_d[pl.ds(srcb, 16)]
            stg_s[pl.ds(r * 16, 16)] = stg_s[pl.ds(srcb, 16)]
            return 0

        lax.fori_loop(0, SB // 16, p1_move, 0)
        return sptr - blocks * SB, done0 + blocks * SB

    sptr, done = lax.fori_loop(0, NCH, p1_chunk, (0, 0))
    fin = pl.multiple_of(mybase + done, SB)
    pltpu.sync_copy(stg_d.at[pl.ds(0, SB)], med_hbm.at[pl.ds(fin, SB)])
    pltpu.sync_copy(stg_s.at[pl.ds(0, SB)], mes_hbm.at[pl.ds(fin, SB)])
    cnt = done + sptr

    nch2 = lax.shift_right_logical(cnt + (MB - 1), 10)

    def start_kv(jb, kbuf, vbuf, sem):
        pltpu.async_copy(k_hbm.at[pend_s.at[pl.ds(jb, GB)]], kbuf, sem)
        pltpu.async_copy(v_hbm.at[pend_s.at[pl.ds(jb, GB)]], vbuf, sem)

    def wait_kv(jb, kbuf, vbuf, sem):
        pltpu.make_async_copy(k_hbm.at[pend_s.at[pl.ds(jb, GB)]], kbuf, sem).wait()
        pltpu.make_async_copy(v_hbm.at[pend_s.at[pl.ds(jb, GB)]], vbuf, sem).wait()

    def pass_body(p, _):
        lo = base + p * PN
        hi = lo + PN
        pltpu.sync_copy(q_hbm.at[pl.ds(lo, PN)], qstage)

        def zero_body(i, _):
            for t in range(64):
                accn[i, pl.ds(t * 16, 16)] = zeros16
            accd[i, pl.ds(0, 16)] = zeros16
            return 0

        lax.fori_loop(0, PN, zero_body, 0)

        def p2_chunk(c2, _):
            cb = c2 * MB
            src = pl.multiple_of(mybase + cb, MB)
            pltpu.sync_copy(med_hbm.at[pl.ds(src, MB)], md)
            pltpu.sync_copy(mes_hbm.at[pl.ds(src, MB)], ms)

            def p2_scan(g, ptr):
                off = g * 16
                d16 = md[pl.ds(off, 16)]
                s16 = ms[pl.ds(off, 16)]
                gv = (cb + off + iota) < cnt
                m = jnp.logical_and(jnp.logical_and(d16 >= lo, d16 < hi), gv)
                c16 = jnp.sum(m.astype(jnp.int32))
                plsc.store_compressed(pend_d.at[pl.ds(ptr, 16)], d16, mask=m)
                plsc.store_compressed(pend_s.at[pl.ds(ptr, 16)], s16, mask=m)
                return ptr + c16

            ptr = lax.fori_loop(0, MB // 16, p2_scan, 0)
            pend_d[pl.ds(ptr, 16)] = zeros16i
            pend_s[pl.ds(ptr, 16)] = zeros16i
            nb = lax.shift_right_logical(ptr + (GB - 1), 3)

            def compute8(jb, kbuf, vbuf):
                ld16 = pend_d[pl.ds(jb, 16)] - lo

                def edge_body(e, _):
                    valid = (jb + e) < ptr
                    oe = (iota == e).astype(jnp.int32)
                    ld = jnp.sum(ld16 * oe)
                    ld = lax.select(valid, ld, 0)
                    av = zeros16
                    for h in range(H):
                        a16 = zeros16
                        for t in range(16):
                            cc = (h * 16 + t) * 16
                            a16 = a16 + qstage[ld, pl.ds(cc, 16)] * kbuf[e, pl.ds(cc, 16)]
                        av = av + oneh[h] * _bflysum(a16, iota)
                    ev = jnp.exp(av * INVSQ) * jnp.where(valid, 1.0, 0.0)
                    plsc.addupdate(accd.at[ld, pl.ds(0, 16)], ev * first4)
                    for h in range(H):
                        evh = _gather16(ev, jnp.full((16,), h, jnp.int32))
                        for t in range(16):
                            cc = (h * 16 + t) * 16
                            plsc.addupdate(accn.at[ld, pl.ds(cc, 16)],
                                           evh * vbuf[e, pl.ds(cc, 16)])
                    return 0

                lax.fori_loop(0, GB, edge_body, 0)

            @pl.when(nb > 0)
            def _():
                start_kv(0, ka, va, sema)

                def ring2(jj, _):
                    j0 = jj * 2
                    j1 = j0 + 1

                    @pl.when(j1 < nb)
                    def _():
                        start_kv(j1 * GB, kb, vb, semb)

                    wait_kv(j0 * GB, ka, va, sema)
                    compute8(j0 * GB, ka, va)

                    @pl.when(j1 < nb)
                    def _():
                        @pl.when(j1 + 1 < nb)
                        def _():
                            start_kv((j1 + 1) * GB, ka, va, sema)

                        wait_kv(j1 * GB, kb, vb, semb)
                        compute8(j1 * GB, kb, vb)

                    return 0

                lax.fori_loop(0, lax.shift_right_logical(nb + 1, 1), ring2, 0)

            return 0

        lax.fori_loop(0, nch2, p2_chunk, 0)
        pltpu.sync_copy(accn, num_hbm.at[pl.ds(lo, PN)])
        pltpu.sync_copy(accd, den_hbm.at[pl.ds(lo, PN)])
        return 0

    lax.fori_loop(0, NPASS, pass_body, 0)


def _edge_phase(q2, k2, v2, edge_index, edge_type):
    ed = jnp.concatenate([edge_index, edge_type.reshape(1, E)], axis=0)
    mesh = plsc.VectorSubcoreMesh(core_axis_name="c", subcore_axis_name="s")
    num, den, _, _ = pl.kernel(
        _edge_body,
        mesh=mesh,
        compiler_params=pltpu.CompilerParams(needs_layout_passes=False),
        out_type=[
            jax.ShapeDtypeStruct((SCN, H * D), jnp.float32),
            jax.ShapeDtypeStruct((SCN, 16), jnp.float32),
            jax.ShapeDtypeStruct((NTILE * EE,), jnp.int32),
            jax.ShapeDtypeStruct((NTILE * EE,), jnp.int32),
        ],
        scratch_types=[
            pltpu.VMEM((3, CE), jnp.int32),
            pltpu.VMEM((CE + 2 * SB + 16,), jnp.int32),
            pltpu.VMEM((CE + 2 * SB + 16,), jnp.int32),
            pltpu.VMEM((MB,), jnp.int32),
            pltpu.VMEM((MB,), jnp.int32),
            pltpu.VMEM((MB + 16,), jnp.int32),
            pltpu.VMEM((MB + 16,), jnp.int32),
            pltpu.VMEM((GB, H * D), jnp.float32),
            pltpu.VMEM((GB, H * D), jnp.float32),
            pltpu.VMEM((GB, H * D), jnp.float32),
            pltpu.VMEM((GB, H * D), jnp.float32),
            pltpu.VMEM((PN, H * D), jnp.float32),
            pltpu.VMEM((PN, H * D), jnp.float32),
            pltpu.VMEM((PN, 16), jnp.float32),
            pltpu.SemaphoreType.DMA,
            pltpu.SemaphoreType.DMA,
        ],
    )(ed, q2, k2, v2)
    return num, den


def _combine_body(num0_ref, den0_ref, s0_ref, num1_ref, den1_ref, s1_ref,
                  x_ref, wgu_ref, wgx_ref, bg_ref, a1_ref, b1_ref, a2_ref,
                  z0_ref, z1_ref, w_ref):
    i = pl.program_id(0)
    x = x_ref[...]
    xg = jnp.dot(x, wgx_ref[...], preferred_element_type=jnp.float32)

    @pl.when(i == 0)
    def _():
        w_ref[...] = jnp.zeros_like(w_ref)

    wacc = jnp.zeros((SH, 2), jnp.float32)
    for r, (num_ref, den_ref, s_ref, z_ref) in enumerate(
            ((num0_ref, den0_ref, s0_ref, z0_ref),
             (num1_ref, den1_ref, s1_ref, z1_ref))):
        num = num_ref[...]
        den = den_ref[...]
        u = jnp.zeros((BLK, D), jnp.float32)
        for h in range(H):
            u = u + num[:, h * D:(h + 1) * D] / (den[:, h:h + 1] + EPS)
        u = u * (1.0 / H) + s_ref[...]
        g = jax.nn.sigmoid(jnp.dot(u, wgu_ref[...], preferred_element_type=jnp.float32)
                           + xg + bg_ref[...])
        z = jnp.tanh(u) * g + x * (1.0 - g)
        z_ref[...] = z
        for sh in range(SH):
            t = jnp.tanh(jnp.dot(z, a1_ref[sh], preferred_element_type=jnp.float32)
                         + b1_ref[sh:sh + 1, :])
            t = jnp.dot(t, a2_ref[sh], preferred_element_type=jnp.float32)
            wacc = wacc + jnp.sum(t) * (jnp.arange(SH)[:, None] == sh) * \
                (jnp.arange(2)[None, :] == r)
    w_ref[...] += wacc


def _combine(num, den, skip, x, Wg, bg, A1, b1, A2):
    Wgu = Wg[:D]
    Wgx = Wg[D:]
    bg2 = bg.reshape(1, D)
    full = lambda *s: pl.BlockSpec(s, lambda i: (0,) * len(s))
    half = lambda r: pl.BlockSpec((BLK, H * D), lambda i, _r=r: (_r * NB + i, 0))
    halfd = lambda r: pl.BlockSpec((BLK, 16), lambda i, _r=r: (_r * NB + i, 0))
    halfs = lambda r: pl.BlockSpec((BLK, D), lambda i, _r=r: (_r * NB + i, 0))
    return pl.pallas_call(
        _combine_body,
        grid=(NB,),
        in_specs=[
            half(0), halfd(0), halfs(0), half(1), halfd(1), halfs(1),
            pl.BlockSpec((BLK, D), lambda i: (i, 0)),
            full(D, D), full(D, D), full(1, D),
            full(SH, D, HID), full(SH, HID), full(SH, HID, 1),
        ],
        out_specs=[
            pl.BlockSpec((BLK, D), lambda i: (i, 0)),
            pl.BlockSpec((BLK, D), lambda i: (i, 0)),
            pl.BlockSpec((SH, 2), lambda i: (0, 0)),
        ],
        out_shape=[
            jax.ShapeDtypeStruct((N, D), jnp.float32),
            jax.ShapeDtypeStruct((N, D), jnp.float32),
            jax.ShapeDtypeStruct((SH, 2), jnp.float32),
        ],
    )(num, den, skip, num, den, skip, x, Wgu, Wgx, bg2, A1, b1, A2)


def _final_body(z0_ref, z1_ref, w_ref, o_ref):
    w = w_ref[...] * (1.0 / N)
    ew = jnp.exp(w - jnp.max(w, axis=1, keepdims=True))
    beta = ew / jnp.sum(ew, axis=1, keepdims=True)
    c0 = jnp.sum(beta[:, 0]) * (1.0 / SH)
    c1 = jnp.sum(beta[:, 1]) * (1.0 / SH)
    o_ref[...] = c0 * z0_ref[...] + c1 * z1_ref[...]


def _final(z0, z1, w):
    return pl.pallas_call(
        _final_body,
        grid=(NB,),
        in_specs=[
            pl.BlockSpec((BLK, D), lambda i: (i, 0)),
            pl.BlockSpec((BLK, D), lambda i: (i, 0)),
            pl.BlockSpec((SH, 2), lambda i: (0, 0)),
        ],
        out_specs=pl.BlockSpec((BLK, D), lambda i: (i, 0)),
        out_shape=jax.ShapeDtypeStruct((N, D), jnp.float32),
    )(z0, z1, w)


def kernel(features, edge_index, edge_type, Wq, bq, Wk, bk, Wv, bv, Ws, bs,
           Wg, bg, A1, b1, A2):
    q2, k2, v2, skip = _projections(features, Wq, bq, Wk, bk, Wv, bv, Ws, bs)
    num, den = _edge_phase(q2, k2, v2, edge_index, edge_type)
    z0, z1, w = _combine(num, den, skip, features, Wg, bg, A1, b1, A2)
    return _final(z0, z1, w)
